# baseline (speedup 1.0000x reference)
import functools

import jax
import jax.numpy as jnp
from jax import lax
from jax.experimental import pallas as pl
from jax.experimental.pallas import tpu as pltpu
from jax.experimental.pallas import tpu_sc as plsc

_NUM_CLASSES = 1000
_BATCH = 16384
_D = 1000
_DPAD = 1024
_DMAIN = 896
_DTAIL = _D - _DMAIN

_NC = 2
_NS = 16
_NW = _NC * _NS
_SBATCH = _BATCH
_BPW = _SBATCH // _NW
_CHUNK = 16
_NCHUNK = _BPW // _CHUNK
_NBUF = 6
_K = 4


def _emb_body(idx_hbm, table_hbm, out_hbm, idx_v, rows_v, tail_v,
              gat_sem, out_sem, tail_sem):
    wid = lax.axis_index("s") * _NC + lax.axis_index("c")
    base = wid * _BPW
    pltpu.sync_copy(idx_hbm.at[wid], idx_v)

    gats = [None] * _NBUF
    outs = [None] * _NBUF
    touts = [None] * _NBUF

    def _repack_tail(b):
        def body(r, carry):
            for j in range(6):
                tail_v[b, r, pl.ds(16 * j, 16)] = rows_v[b, r, pl.ds(_DMAIN + 16 * j, 16)]
            tail_v[b, r, pl.ds(_DTAIL - 16, 16)] = rows_v[b, r, pl.ds(_DMAIN + _DTAIL - 16, 16)]
            return carry
        lax.fori_loop(0, _CHUNK, body, 0)

    for g in range(_NCHUNK + _K):
        if g < _NCHUNK:
            b = g % _NBUF
            if outs[b] is not None:
                outs[b].wait()
                touts[b].wait()
            gats[b] = pltpu.async_copy(
                table_hbm.at[idx_v.at[g]],
                rows_v.at[b],
                gat_sem.at[b],
            )
        if g >= _K:
            pb = (g - _K) % _NBUF
            row0 = base + (g - _K) * _CHUNK
            gats[pb].wait()
            outs[pb] = pltpu.async_copy(
                rows_v.at[pb, :, pl.ds(0, _DMAIN)],
                out_hbm.at[pl.ds(row0, _CHUNK), pl.ds(0, _DMAIN)],
                out_sem.at[pb],
            )
            _repack_tail(pb)
            touts[pb] = pltpu.async_copy(
                tail_v.at[pb],
                out_hbm.at[pl.ds(row0, _CHUNK), pl.ds(_DMAIN, _DTAIL)],
                tail_sem.at[pb],
            )
    for b in range(_NBUF):
        if outs[b] is not None:
            outs[b].wait()
            touts[b].wait()


@functools.partial(
    pl.kernel,
    mesh=plsc.VectorSubcoreMesh(core_axis_name="c", subcore_axis_name="s"),
    out_type=jax.ShapeDtypeStruct((_SBATCH, _D), jnp.float32),
    scratch_types=[
        pltpu.VMEM((_NCHUNK, _CHUNK), jnp.int32),
        pltpu.VMEM((_NBUF, _CHUNK, _DPAD), jnp.float32),
        pltpu.VMEM((_NBUF, _CHUNK, _DTAIL), jnp.float32),
        pltpu.SemaphoreType.DMA((_NBUF,)),
        pltpu.SemaphoreType.DMA((_NBUF,)),
        pltpu.SemaphoreType.DMA((_NBUF,)),
    ],
)
def _emb(idx_hbm, table_hbm, out_hbm, idx_v, rows_v, tail_v,
         gat_sem, out_sem, tail_sem):
    _emb_body(idx_hbm, table_hbm, out_hbm, idx_v, rows_v, tail_v,
              gat_sem, out_sem, tail_sem)


def kernel(knowledge, embedding_weight):
    idx = knowledge.astype(jnp.int32).reshape(_NW, _NCHUNK, _CHUNK)
    table = jnp.pad(embedding_weight, ((0, 0), (0, _DPAD - _D)))
    return _emb(idx, table)


# speedup vs baseline: 1.0115x; 1.0115x over previous
import functools

import jax
import jax.numpy as jnp
from jax import lax
from jax.experimental import pallas as pl
from jax.experimental.pallas import tpu as pltpu
from jax.experimental.pallas import tpu_sc as plsc

_NUM_CLASSES = 1000
_BATCH = 16384
_D = 1000
_DPAD = 1024
_DMAIN = 896
_DTAIL = _D - _DMAIN

_NC = 2
_NS = 16
_NW = _NC * _NS
_SBATCH = _BATCH
_BPW = _SBATCH // _NW
_CHUNK = 32
_NCHUNK = _BPW // _CHUNK
_NBUF = 3
_K = 2


def _emb_body(idx_hbm, table_hbm, out_hbm, idx_v, rows_v, tail_v,
              gat_sem, out_sem, tail_sem):
    wid = lax.axis_index("s") * _NC + lax.axis_index("c")
    base = wid * _BPW
    pltpu.sync_copy(idx_hbm.at[wid], idx_v)

    gats = [None] * _NBUF
    outs = [None] * _NBUF
    touts = [None] * _NBUF

    def _repack_tail(b):
        def body(r, carry):
            for j in range(6):
                tail_v[b, r, pl.ds(16 * j, 16)] = rows_v[b, r, pl.ds(_DMAIN + 16 * j, 16)]
            tail_v[b, r, pl.ds(_DTAIL - 16, 16)] = rows_v[b, r, pl.ds(_DMAIN + _DTAIL - 16, 16)]
            return carry
        lax.fori_loop(0, _CHUNK, body, 0)

    for g in range(_NCHUNK + _K):
        if g < _NCHUNK:
            b = g % _NBUF
            if outs[b] is not None:
                outs[b].wait()
                touts[b].wait()
            gats[b] = pltpu.async_copy(
                table_hbm.at[idx_v.at[g]],
                rows_v.at[b],
                gat_sem.at[b],
            )
        if g >= _K:
            pb = (g - _K) % _NBUF
            row0 = base + (g - _K) * _CHUNK
            gats[pb].wait()
            outs[pb] = pltpu.async_copy(
                rows_v.at[pb, :, pl.ds(0, _DMAIN)],
                out_hbm.at[pl.ds(row0, _CHUNK), pl.ds(0, _DMAIN)],
                out_sem.at[pb],
            )
            _repack_tail(pb)
            touts[pb] = pltpu.async_copy(
                tail_v.at[pb],
                out_hbm.at[pl.ds(row0, _CHUNK), pl.ds(_DMAIN, _DTAIL)],
                tail_sem.at[pb],
            )
    for b in range(_NBUF):
        if outs[b] is not None:
            outs[b].wait()
            touts[b].wait()


@functools.partial(
    pl.kernel,
    mesh=plsc.VectorSubcoreMesh(core_axis_name="c", subcore_axis_name="s"),
    out_type=jax.ShapeDtypeStruct((_SBATCH, _D), jnp.float32),
    scratch_types=[
        pltpu.VMEM((_NCHUNK, _CHUNK), jnp.int32),
        pltpu.VMEM((_NBUF, _CHUNK, _DPAD), jnp.float32),
        pltpu.VMEM((_NBUF, _CHUNK, _DTAIL), jnp.float32),
        pltpu.SemaphoreType.DMA((_NBUF,)),
        pltpu.SemaphoreType.DMA((_NBUF,)),
        pltpu.SemaphoreType.DMA((_NBUF,)),
    ],
)
def _emb(idx_hbm, table_hbm, out_hbm, idx_v, rows_v, tail_v,
         gat_sem, out_sem, tail_sem):
    _emb_body(idx_hbm, table_hbm, out_hbm, idx_v, rows_v, tail_v,
              gat_sem, out_sem, tail_sem)


def kernel(knowledge, embedding_weight):
    idx = knowledge.astype(jnp.int32).reshape(_NW, _NCHUNK, _CHUNK)
    table = jnp.pad(embedding_weight, ((0, 0), (0, _DPAD - _D)))
    return _emb(idx, table)
